# baseline (speedup 1.0000x reference)
import functools

import jax
import jax.numpy as jnp
from jax import lax
from jax.experimental import pallas as pl
from jax.experimental.pallas import tpu as pltpu
from jax.experimental.pallas import tpu_sc as plsc

HIDDEN = 64
BATCH = 4096
NUM_CORES = 2
NUM_SUBCORES = 16
LANES = 16
NUM_WORKERS = NUM_CORES * NUM_SUBCORES
BPW = BATCH // NUM_WORKERS
GROUPS = BPW // LANES
VPR = HIDDEN // LANES


@functools.partial(
    pl.kernel,
    mesh=plsc.VectorSubcoreMesh(core_axis_name="c", subcore_axis_name="s"),
    out_type=jax.ShapeDtypeStruct((BATCH,), jnp.float32),
    compiler_params=pltpu.CompilerParams(
        needs_layout_passes=False, use_tc_tiling_on_sc=False),
    scratch_types=[
        pltpu.VMEM((BPW,), jnp.int32),
        pltpu.VMEM((BPW,), jnp.int32),
        pltpu.VMEM((BPW, HIDDEN), jnp.float32),
        pltpu.VMEM((BPW, HIDDEN), jnp.float32),
        pltpu.VMEM((BPW,), jnp.float32),
        pltpu.SemaphoreType.DMA,
        pltpu.SemaphoreType.DMA,
    ],
)
def _mf_sc(uidx_hbm, iidx_hbm, ut_hbm, it_hbm, out_hbm,
           uidx_v, iidx_v, urows_v, irows_v, res_v, usem, isem):
    wid = lax.axis_index("s") * NUM_CORES + lax.axis_index("c")
    base = wid * BPW

    pltpu.sync_copy(uidx_hbm.at[pl.ds(base, BPW)], uidx_v)
    pltpu.sync_copy(iidx_hbm.at[pl.ds(base, BPW)], iidx_v)
    ucp = pltpu.async_copy(ut_hbm.at[uidx_v], urows_v, usem)
    icp = pltpu.async_copy(it_hbm.at[iidx_v], irows_v, isem)
    ucp.wait()
    icp.wait()

    lane_iota = lax.iota(jnp.int32, LANES)
    for g in range(GROUPS):
        acc = jnp.zeros((LANES,), jnp.float32)
        for r in range(LANES):
            row = g * LANES + r
            p = (urows_v[row, pl.ds(0, LANES)] *
                 irows_v[row, pl.ds(0, LANES)])
            for j in range(1, VPR):
                p = p + (urows_v[row, pl.ds(j * LANES, LANES)] *
                         irows_v[row, pl.ds(j * LANES, LANES)])
            acc = jnp.where(lane_iota == r, jnp.sum(p), acc)
        res_v[pl.ds(g * LANES, LANES)] = 1.0 / (1.0 + jnp.exp(-acc))

    pltpu.sync_copy(res_v, out_hbm.at[pl.ds(base, BPW)])


def kernel(x, user_table, item_table):
    uidx = x[:, 0].astype(jnp.int32)
    iidx = x[:, 1].astype(jnp.int32)
    return _mf_sc(uidx, iidx, user_table, item_table)


# speedup vs baseline: 1.0022x; 1.0022x over previous
import functools

import jax
import jax.numpy as jnp
from jax import lax
from jax.experimental import pallas as pl
from jax.experimental.pallas import tpu as pltpu
from jax.experimental.pallas import tpu_sc as plsc

HIDDEN = 64
BATCH = 4096
NUM_CORES = 2
NUM_SUBCORES = 16
LANES = 16
NUM_WORKERS = NUM_CORES * NUM_SUBCORES
BPW = BATCH // NUM_WORKERS
GROUPS = BPW // LANES
VPR = HIDDEN // LANES
ACC_PITCH = LANES + 1


@functools.partial(
    pl.kernel,
    mesh=plsc.VectorSubcoreMesh(core_axis_name="c", subcore_axis_name="s"),
    out_type=jax.ShapeDtypeStruct((BATCH,), jnp.float32),
    compiler_params=pltpu.CompilerParams(
        needs_layout_passes=False, use_tc_tiling_on_sc=False),
    scratch_types=[
        pltpu.VMEM((BPW,), jnp.int32),
        pltpu.VMEM((BPW,), jnp.int32),
        pltpu.VMEM((BPW, HIDDEN), jnp.float32),
        pltpu.VMEM((BPW, HIDDEN), jnp.float32),
        pltpu.VMEM((BPW, ACC_PITCH), jnp.float32),
        pltpu.VMEM((BPW,), jnp.float32),
        pltpu.SemaphoreType.DMA,
        pltpu.SemaphoreType.DMA,
    ],
)
def _mf_sc(uidx_hbm, iidx_hbm, ut_hbm, it_hbm, out_hbm,
           uidx_v, iidx_v, ubuf_v, ibuf_v, acc_v, res_v, usem, isem):
    wid = lax.axis_index("s") * NUM_CORES + lax.axis_index("c")
    base = wid * BPW

    pltpu.sync_copy(uidx_hbm.at[pl.ds(base, BPW)], uidx_v)
    pltpu.sync_copy(iidx_hbm.at[pl.ds(base, BPW)], iidx_v)

    ucp = pltpu.async_copy(ut_hbm.at[uidx_v], ubuf_v, usem)
    icp = pltpu.async_copy(it_hbm.at[iidx_v], ibuf_v, isem)
    ucp.wait()
    icp.wait()

    def row_body(r, _):
        acc = ubuf_v[r, pl.ds(0, LANES)] * ibuf_v[r, pl.ds(0, LANES)]
        for j in range(1, VPR):
            acc = acc + (ubuf_v[r, pl.ds(j * LANES, LANES)]
                         * ibuf_v[r, pl.ds(j * LANES, LANES)])
        acc_v[r, pl.ds(0, LANES)] = acc
        return _

    lax.fori_loop(0, BPW, row_body, None)

    iota16 = lax.broadcasted_iota(jnp.int32, (LANES,), 0)
    for g in range(GROUPS):
        rows = iota16 + (g * LANES)
        tot = plsc.load_gather(acc_v, [rows, jnp.zeros((LANES,), jnp.int32)])
        for j in range(1, LANES):
            tot = tot + plsc.load_gather(
                acc_v, [rows, jnp.full((LANES,), j, jnp.int32)])
        res_v[pl.ds(g * LANES, LANES)] = 1.0 / (1.0 + jnp.exp(-tot))

    pltpu.sync_copy(res_v, out_hbm.at[pl.ds(base, BPW)])


def kernel(x, user_table, item_table):
    uidx = x[:, 0].astype(jnp.int32)
    iidx = x[:, 1].astype(jnp.int32)
    return _mf_sc(uidx, iidx, user_table, item_table)
